# baseline (speedup 1.0000x reference)
import functools

import jax
import jax.numpy as jnp
from jax import lax
from jax.experimental import pallas as pl
from jax.experimental.pallas import tpu as pltpu
from jax.experimental.pallas import tpu_sc as plsc

_NC = 2
_NS = 16
_NW = _NC * _NS

_RING = 4
_E_CHUNK = 200

_N_GRID = 25


def _sc_edges_body(edges_hbm, out_edges_hbm, eb0, eb1, eb2, eb3,
                   sems_in, sems_out):
    wid = lax.axis_index("s") * _NC + lax.axis_index("c")
    e_rows = edges_hbm.shape[0] // _NW
    e_base = pl.multiple_of(wid * e_rows, 8)
    n_chunks = e_rows // _E_CHUNK
    ebufs = (eb0, eb1, eb2, eb3)

    def in_copy(i):
        return pltpu.make_async_copy(
            edges_hbm.at[pl.ds(e_base + i * _E_CHUNK, _E_CHUNK), :],
            ebufs[i % _RING], sems_in.at[i % _RING])

    def out_copy(i):
        return pltpu.make_async_copy(
            ebufs[i % _RING],
            out_edges_hbm.at[pl.ds(e_base + i * _E_CHUNK, _E_CHUNK), :],
            sems_out.at[i % _RING])

    for i in range(min(_RING, n_chunks)):
        in_copy(i).start()
    for i in range(n_chunks):
        in_copy(i).wait()
        out_copy(i).start()
        if i + _RING < n_chunks:
            out_copy(i).wait()
            in_copy(i + _RING).start()
    for i in range(max(n_chunks - _RING, 0), n_chunks):
        out_copy(i).wait()


def _tc_nodes_body(nodes_ref, out_nodes_ref):
    out_nodes_ref[...] = nodes_ref[...]


def kernel(node_latents_from, node_latents_to, edge_latents, edge_index,
           receivers_count):
    del node_latents_from, edge_index, receivers_count
    n_nodes, d_feat = node_latents_to.shape
    node_rows = n_nodes // _N_GRID

    mesh = plsc.VectorSubcoreMesh(
        core_axis_name="c", subcore_axis_name="s",
        num_cores=_NC, num_subcores=_NS)
    new_edges = functools.partial(
        pl.kernel,
        out_type=jax.ShapeDtypeStruct(edge_latents.shape, edge_latents.dtype),
        mesh=mesh,
        scratch_types=[
            pltpu.VMEM((_E_CHUNK, 16), jnp.float32),
            pltpu.VMEM((_E_CHUNK, 16), jnp.float32),
            pltpu.VMEM((_E_CHUNK, 16), jnp.float32),
            pltpu.VMEM((_E_CHUNK, 16), jnp.float32),
            pltpu.SemaphoreType.DMA((_RING,)),
            pltpu.SemaphoreType.DMA((_RING,)),
        ],
    )(_sc_edges_body)(edge_latents)

    new_nodes = pl.pallas_call(
        _tc_nodes_body,
        grid=(_N_GRID,),
        out_shape=jax.ShapeDtypeStruct(node_latents_to.shape,
                                       node_latents_to.dtype),
        in_specs=[pl.BlockSpec((node_rows, d_feat), lambda i: (i, 0))],
        out_specs=pl.BlockSpec((node_rows, d_feat), lambda i: (i, 0)),
    )(node_latents_to)
    return (new_nodes, new_edges)
